# Initial kernel scaffold; baseline (speedup 1.0000x reference)
import jax
import jax.numpy as jnp
from jax.experimental import pallas as pl


def kernel(x, edge_index, batch, W1, b1, W2, b2, Wlin, blin):
    raise NotImplementedError("write your pallas kernel here")


# speedup vs baseline: 10.2294x; 10.2294x over previous
import jax
import jax.numpy as jnp
from jax import lax
from jax.experimental import pallas as pl
from jax.experimental.pallas import tpu as pltpu
from jax.experimental.pallas import tpu_sc as plsc

N = 10000
E = 320000
D = 128
H = 128
C = 40

NC = 2
NS = 16
NW = NC * NS
EPW = E // NW
CHUNK = 80
NCHUNK = EPW // CHUNK
NP = 10240
ROWS_PT = NP // NS
DEGW = 128

_sc_mesh = plsc.VectorSubcoreMesh(
    core_axis_name="c", subcore_axis_name="s", num_cores=NC, num_subcores=NS)


def _deg_body(dst_hbm, ones_hbm, zeros_hbm, out_hbm, dst_v, ones_v, deg_sh):
    cid = lax.axis_index("c")
    sid = lax.axis_index("s")
    wid = sid * NC + cid
    r0 = sid * ROWS_PT
    pltpu.sync_copy(zeros_hbm.at[pl.ds(r0, ROWS_PT)],
                    deg_sh.at[pl.ds(r0, ROWS_PT)])
    pltpu.sync_copy(ones_hbm, ones_v)
    plsc.subcore_barrier()
    base = wid * EPW

    def step(j, carry):
        pltpu.sync_copy(dst_hbm.at[pl.ds(base + j * CHUNK, CHUNK)], dst_v)
        pltpu.sync_copy(ones_v, deg_sh.at[dst_v], add=True)
        return carry

    lax.fori_loop(0, NCHUNK, step, 0)
    plsc.subcore_barrier()
    pltpu.sync_copy(deg_sh.at[pl.ds(r0, ROWS_PT)],
                    out_hbm.at[cid, pl.ds(r0, ROWS_PT)])


_deg_call = pl.kernel(
    _deg_body,
    out_type=jax.ShapeDtypeStruct((NC, NP, DEGW), jnp.float32),
    mesh=_sc_mesh,
    scratch_types=[
        pltpu.VMEM((CHUNK,), jnp.int32),
        pltpu.VMEM((CHUNK, DEGW), jnp.float32),
        pltpu.VMEM_SHARED((NP, DEGW), jnp.float32),
    ],
)


def _prop_body(hp_hbm, src_hbm, dst_hbm, zeros_hbm, out_hbm,
               src_v, dst_v, rows_v, agg_sh, sem):
    cid = lax.axis_index("c")
    sid = lax.axis_index("s")
    wid = sid * NC + cid
    r0 = sid * ROWS_PT
    pltpu.sync_copy(zeros_hbm.at[pl.ds(r0, ROWS_PT)],
                    agg_sh.at[pl.ds(r0, ROWS_PT)])
    plsc.subcore_barrier()
    base = wid * EPW

    def step(j, carry):
        s = base + j * CHUNK
        pltpu.sync_copy(src_hbm.at[pl.ds(s, CHUNK)], src_v)
        pltpu.sync_copy(dst_hbm.at[pl.ds(s, CHUNK)], dst_v)
        pltpu.async_copy(hp_hbm.at[src_v], rows_v, sem).wait()
        pltpu.sync_copy(rows_v, agg_sh.at[dst_v], add=True)
        return carry

    lax.fori_loop(0, NCHUNK, step, 0)
    plsc.subcore_barrier()
    pltpu.sync_copy(agg_sh.at[pl.ds(r0, ROWS_PT)],
                    out_hbm.at[cid, pl.ds(r0, ROWS_PT)])


_prop_call = pl.kernel(
    _prop_body,
    out_type=jax.ShapeDtypeStruct((NC, NP, H), jnp.float32),
    mesh=_sc_mesh,
    scratch_types=[
        pltpu.VMEM((CHUNK,), jnp.int32),
        pltpu.VMEM((CHUNK,), jnp.int32),
        pltpu.VMEM((CHUNK, H), jnp.float32),
        pltpu.VMEM_SHARED((NP, H), jnp.float32),
        pltpu.SemaphoreType.DMA,
    ],
)

_RB = 2000
_GRID = N // _RB


def _scale_body(dp_ref, x_ref, w_ref, dis_ref, h_ref):
    dp = dp_ref[...]
    deg = dp[0, :, 0:1] + dp[1, :, 0:1]
    dis = jnp.where(deg > 0, 1.0 / jnp.sqrt(jnp.maximum(deg, 1e-12)), 0.0)
    disb = jnp.broadcast_to(dis, (_RB, H))
    h = jnp.dot(x_ref[...], w_ref[...], preferred_element_type=jnp.float32)
    dis_ref[...] = disb
    h_ref[...] = h * disb


_scale_call = pl.pallas_call(
    _scale_body,
    grid=(_GRID,),
    in_specs=[
        pl.BlockSpec((NC, _RB, DEGW), lambda i: (0, i, 0)),
        pl.BlockSpec((_RB, D), lambda i: (i, 0)),
        pl.BlockSpec((D, H), lambda i: (0, 0)),
    ],
    out_specs=[
        pl.BlockSpec((_RB, H), lambda i: (i, 0)),
        pl.BlockSpec((_RB, H), lambda i: (i, 0)),
    ],
    out_shape=[
        jax.ShapeDtypeStruct((N, H), jnp.float32),
        jax.ShapeDtypeStruct((N, H), jnp.float32),
    ],
)


def _mid_body(p_ref, dis_ref, b_ref, w_ref, out_ref):
    p = p_ref[...]
    dis = dis_ref[...]
    h1 = jnp.maximum(dis * (p[0] + p[1]) + b_ref[...], 0.0)
    out_ref[...] = jnp.dot(
        h1, w_ref[...], preferred_element_type=jnp.float32) * dis


_mid_call = pl.pallas_call(
    _mid_body,
    grid=(_GRID,),
    in_specs=[
        pl.BlockSpec((NC, _RB, H), lambda i: (0, i, 0)),
        pl.BlockSpec((_RB, H), lambda i: (i, 0)),
        pl.BlockSpec((1, H), lambda i: (0, 0)),
        pl.BlockSpec((H, H), lambda i: (0, 0)),
    ],
    out_specs=pl.BlockSpec((_RB, H), lambda i: (i, 0)),
    out_shape=jax.ShapeDtypeStruct((N, H), jnp.float32),
)


def _head_body(p_ref, dis_ref, b_ref, w_ref, bl_ref, out_ref):
    p = p_ref[...]
    h2 = jnp.maximum(dis_ref[...] * (p[0] + p[1]) + b_ref[...], 0.0)
    logits = jnp.dot(
        h2, w_ref[...], preferred_element_type=jnp.float32) + bl_ref[...]
    m = jnp.max(logits, axis=-1, keepdims=True)
    sh = logits - m
    lse = jnp.log(jnp.sum(jnp.exp(sh), axis=-1, keepdims=True))
    out_ref[...] = sh - lse


_head_call = pl.pallas_call(
    _head_body,
    grid=(_GRID,),
    in_specs=[
        pl.BlockSpec((NC, _RB, H), lambda i: (0, i, 0)),
        pl.BlockSpec((_RB, H), lambda i: (i, 0)),
        pl.BlockSpec((1, H), lambda i: (0, 0)),
        pl.BlockSpec((H, C), lambda i: (0, 0)),
        pl.BlockSpec((1, C), lambda i: (0, 0)),
    ],
    out_specs=pl.BlockSpec((_RB, C), lambda i: (i, 0)),
    out_shape=jax.ShapeDtypeStruct((N, C), jnp.float32),
)


def kernel(x, edge_index, batch, W1, b1, W2, b2, Wlin, blin):
    src = edge_index[0]
    dst = edge_index[1]
    zdeg = jnp.zeros((NP, DEGW), jnp.float32)
    zagg = jnp.zeros((NP, H), jnp.float32)
    ones = jnp.ones((CHUNK, DEGW), jnp.float32)

    deg_parts = _deg_call(dst, ones, zdeg)
    dis2d, h1p = _scale_call(deg_parts, x, W1)
    p1 = _prop_call(h1p, src, dst, zagg)
    h2p = _mid_call(p1, dis2d, b1.reshape(1, H), W2)
    p2 = _prop_call(h2p, src, dst, zagg)
    return _head_call(p2, dis2d, b2.reshape(1, H), Wlin, blin.reshape(1, C))


# speedup vs baseline: 19.9252x; 1.9478x over previous
import jax
import jax.numpy as jnp
from jax import lax
from jax.experimental import pallas as pl
from jax.experimental.pallas import tpu as pltpu
from jax.experimental.pallas import tpu_sc as plsc

N = 10000
E = 320000
D = 128
H = 128
C = 40

NC = 2
NS = 16
NW = NC * NS
EPW = E // NW
CHUNK = 80
NCHUNK = EPW // CHUNK
NP = 10240
ROWS_PT = NP // NS
DEGW = 128

_sc_mesh = plsc.VectorSubcoreMesh(
    core_axis_name="c", subcore_axis_name="s", num_cores=NC, num_subcores=NS)


KBUF = 5
NGROUP = NCHUNK // KBUF
PKBUF = 4
PNGROUP = NCHUNK // PKBUF


def _deg_body(dst3_hbm, ones_hbm, zeros_hbm, out_hbm, dst_v, ones_v,
              deg_sh, ssem):
    cid = lax.axis_index("c")
    sid = lax.axis_index("s")
    wid = sid * NC + cid
    r0 = sid * ROWS_PT
    pltpu.sync_copy(zeros_hbm, deg_sh.at[pl.ds(r0, ROWS_PT)])
    pltpu.sync_copy(ones_hbm, ones_v)
    pltpu.sync_copy(dst3_hbm.at[wid], dst_v)
    plsc.subcore_barrier()

    def group(g, carry):
        descs = []
        for b in range(KBUF):
            c = g * KBUF + b
            descs.append(pltpu.async_copy(
                ones_v, deg_sh.at[dst_v.at[c]], ssem.at[b], add=True))
        for d in descs:
            d.wait()
        return carry

    lax.fori_loop(0, NGROUP, group, 0)
    plsc.subcore_barrier()
    pltpu.sync_copy(deg_sh.at[pl.ds(r0, ROWS_PT)],
                    out_hbm.at[cid, pl.ds(r0, ROWS_PT)])


_deg_call = pl.kernel(
    _deg_body,
    out_type=jax.ShapeDtypeStruct((NC, NP, DEGW), jnp.float32),
    mesh=_sc_mesh,
    scratch_types=[
        pltpu.VMEM((NCHUNK, CHUNK), jnp.int32),
        pltpu.VMEM((CHUNK, DEGW), jnp.float32),
        pltpu.VMEM_SHARED((NP, DEGW), jnp.float32),
        pltpu.SemaphoreType.DMA((KBUF,)),
    ],
)


def _prop_body(hp_hbm, src_hbm, dst_hbm, zeros_hbm, out_hbm,
               agg_sh, gsem, ssem, isem, jsem,
               gi0, gi1, gi2, gi3, di0, di1, di2, di3,
               rv0, rv1, rv2, rv3):
    gidx = (gi0, gi1, gi2, gi3)
    didx = (di0, di1, di2, di3)
    rows = (rv0, rv1, rv2, rv3)
    cid = lax.axis_index("c")
    sid = lax.axis_index("s")
    wid = sid * NC + cid
    r0 = sid * ROWS_PT
    base = wid * EPW
    pltpu.sync_copy(zeros_hbm, agg_sh.at[pl.ds(r0, ROWS_PT)])
    plsc.subcore_barrier()

    def _fetch(c, b):
        pltpu.async_copy(
            src_hbm.at[pl.ds(base + c * CHUNK, CHUNK)], gidx[b], isem.at[b])
        pltpu.async_copy(
            dst_hbm.at[pl.ds(base + c * CHUNK, CHUNK)], didx[b], jsem.at[b])

    def _fetch_wait(c, b):
        pltpu.make_async_copy(
            src_hbm.at[pl.ds(base + c * CHUNK, CHUNK)], gidx[b],
            isem.at[b]).wait()
        pltpu.make_async_copy(
            dst_hbm.at[pl.ds(base + c * CHUNK, CHUNK)], didx[b],
            jsem.at[b]).wait()

    def _gather(b):
        return pltpu.async_copy(hp_hbm.at[gidx[b]], rows[b], gsem.at[b])

    def _scatter(b):
        return pltpu.async_copy(rows[b], agg_sh.at[didx[b]], ssem.at[b],
                                add=True)

    for b in range(PKBUF):
        _fetch(b, b)

    def group(g, carry):
        gd = []
        for b in range(PKBUF):
            c = g * PKBUF + b
            _fetch_wait(c, b)
            gd.append(_gather(b))
        sd = []
        for b in range(PKBUF):
            gd[b].wait()
            sd.append(_scatter(b))
        for b in range(PKBUF):
            c = g * PKBUF + b
            sd[b].wait()
            _fetch(lax.rem(c + PKBUF, NCHUNK), b)
        return carry

    lax.fori_loop(0, PNGROUP, group, 0)
    _fetch_wait(PNGROUP * PKBUF, 0)
    _gather(0).wait()
    _scatter(0).wait()
    for b in range(1, PKBUF):
        _fetch_wait(b - 1, b)
    plsc.subcore_barrier()
    pltpu.sync_copy(agg_sh.at[pl.ds(r0, ROWS_PT)],
                    out_hbm.at[cid, pl.ds(r0, ROWS_PT)])


_prop_call = pl.kernel(
    _prop_body,
    out_type=jax.ShapeDtypeStruct((NC, NP, H), jnp.float32),
    mesh=_sc_mesh,
    scratch_types=[
        pltpu.VMEM_SHARED((NP, H), jnp.float32),
        pltpu.SemaphoreType.DMA((PKBUF,)),
        pltpu.SemaphoreType.DMA((PKBUF,)),
        pltpu.SemaphoreType.DMA((PKBUF,)),
        pltpu.SemaphoreType.DMA((PKBUF,)),
    ] + [pltpu.VMEM((CHUNK,), jnp.int32) for _ in range(2 * PKBUF)]
      + [pltpu.VMEM((CHUNK, H), jnp.float32) for _ in range(PKBUF)],
)

_RB = 2000
_GRID = N // _RB


def _scale_body(dp_ref, x_ref, w_ref, dis_ref, h_ref):
    dp = dp_ref[...]
    deg = dp[0, :, 0:1] + dp[1, :, 0:1]
    dis = jnp.where(deg > 0, 1.0 / jnp.sqrt(jnp.maximum(deg, 1e-12)), 0.0)
    disb = jnp.broadcast_to(dis, (_RB, H))
    h = jnp.dot(x_ref[...], w_ref[...], preferred_element_type=jnp.float32)
    dis_ref[...] = disb
    h_ref[...] = h * disb


_scale_call = pl.pallas_call(
    _scale_body,
    grid=(_GRID,),
    in_specs=[
        pl.BlockSpec((NC, _RB, DEGW), lambda i: (0, i, 0)),
        pl.BlockSpec((_RB, D), lambda i: (i, 0)),
        pl.BlockSpec((D, H), lambda i: (0, 0)),
    ],
    out_specs=[
        pl.BlockSpec((_RB, H), lambda i: (i, 0)),
        pl.BlockSpec((_RB, H), lambda i: (i, 0)),
    ],
    out_shape=[
        jax.ShapeDtypeStruct((N, H), jnp.float32),
        jax.ShapeDtypeStruct((N, H), jnp.float32),
    ],
)


def _mid_body(p_ref, dis_ref, b_ref, w_ref, out_ref):
    p = p_ref[...]
    dis = dis_ref[...]
    h1 = jnp.maximum(dis * (p[0] + p[1]) + b_ref[...], 0.0)
    out_ref[...] = jnp.dot(
        h1, w_ref[...], preferred_element_type=jnp.float32) * dis


_mid_call = pl.pallas_call(
    _mid_body,
    grid=(_GRID,),
    in_specs=[
        pl.BlockSpec((NC, _RB, H), lambda i: (0, i, 0)),
        pl.BlockSpec((_RB, H), lambda i: (i, 0)),
        pl.BlockSpec((1, H), lambda i: (0, 0)),
        pl.BlockSpec((H, H), lambda i: (0, 0)),
    ],
    out_specs=pl.BlockSpec((_RB, H), lambda i: (i, 0)),
    out_shape=jax.ShapeDtypeStruct((N, H), jnp.float32),
)


def _head_body(p_ref, dis_ref, b_ref, w_ref, bl_ref, out_ref):
    p = p_ref[...]
    h2 = jnp.maximum(dis_ref[...] * (p[0] + p[1]) + b_ref[...], 0.0)
    logits = jnp.dot(
        h2, w_ref[...], preferred_element_type=jnp.float32) + bl_ref[...]
    m = jnp.max(logits, axis=-1, keepdims=True)
    sh = logits - m
    lse = jnp.log(jnp.sum(jnp.exp(sh), axis=-1, keepdims=True))
    out_ref[...] = sh - lse


_head_call = pl.pallas_call(
    _head_body,
    grid=(_GRID,),
    in_specs=[
        pl.BlockSpec((NC, _RB, H), lambda i: (0, i, 0)),
        pl.BlockSpec((_RB, H), lambda i: (i, 0)),
        pl.BlockSpec((1, H), lambda i: (0, 0)),
        pl.BlockSpec((H, C), lambda i: (0, 0)),
        pl.BlockSpec((1, C), lambda i: (0, 0)),
    ],
    out_specs=pl.BlockSpec((_RB, C), lambda i: (i, 0)),
    out_shape=jax.ShapeDtypeStruct((N, C), jnp.float32),
)


def kernel(x, edge_index, batch, W1, b1, W2, b2, Wlin, blin):
    src = edge_index[0]
    dst = edge_index[1]
    dst3 = dst.reshape(NW, NCHUNK, CHUNK)
    zdeg = jnp.zeros((ROWS_PT, DEGW), jnp.float32)
    zagg = jnp.zeros((ROWS_PT, H), jnp.float32)
    ones = jnp.ones((CHUNK, DEGW), jnp.float32)

    deg_parts = _deg_call(dst3, ones, zdeg)
    dis2d, h1p = _scale_call(deg_parts, x, W1)
    p1 = _prop_call(h1p, src, dst, zagg)
    h2p = _mid_call(p1, dis2d, b1.reshape(1, H), W2)
    p2 = _prop_call(h2p, src, dst, zagg)
    return _head_call(p2, dis2d, b2.reshape(1, H), Wlin, blin.reshape(1, C))


# speedup vs baseline: 22.4867x; 1.1286x over previous
import jax
import jax.numpy as jnp
from jax import lax
from jax.experimental import pallas as pl
from jax.experimental.pallas import tpu as pltpu
from jax.experimental.pallas import tpu_sc as plsc

N = 10000
E = 320000
D = 128
H = 128
C = 40

NC = 2
NS = 16
NW = NC * NS
EPW = E // NW
CHUNK = 80
NCHUNK = EPW // CHUNK
NP = 10240
ROWS_PT = NP // NS
DEGW = 128

_sc_mesh = plsc.VectorSubcoreMesh(
    core_axis_name="c", subcore_axis_name="s", num_cores=NC, num_subcores=NS)


KBUF = 5
NGROUP = NCHUNK // KBUF
PKBUF = 4
PNGROUP = NCHUNK // PKBUF


def _deg_body(dst3_hbm, ones_hbm, zeros_hbm, out_hbm, dst_v, ones_v,
              deg_sh, ssem):
    cid = lax.axis_index("c")
    sid = lax.axis_index("s")
    wid = sid * NC + cid
    r0 = sid * ROWS_PT
    pltpu.sync_copy(zeros_hbm, deg_sh.at[pl.ds(r0, ROWS_PT)])
    pltpu.sync_copy(ones_hbm, ones_v)
    pltpu.sync_copy(dst3_hbm.at[wid], dst_v)
    plsc.subcore_barrier()

    def group(g, carry):
        descs = []
        for b in range(KBUF):
            c = g * KBUF + b
            descs.append(pltpu.async_copy(
                ones_v, deg_sh.at[dst_v.at[c]], ssem.at[b], add=True))
        for d in descs:
            d.wait()
        return carry

    lax.fori_loop(0, NGROUP, group, 0)
    plsc.subcore_barrier()
    pltpu.sync_copy(deg_sh.at[pl.ds(r0, ROWS_PT)],
                    out_hbm.at[cid, pl.ds(r0, ROWS_PT)])


_deg_call = pl.kernel(
    _deg_body,
    out_type=jax.ShapeDtypeStruct((NC, NP, DEGW), jnp.float32),
    mesh=_sc_mesh,
    scratch_types=[
        pltpu.VMEM((NCHUNK, CHUNK), jnp.int32),
        pltpu.VMEM((CHUNK, DEGW), jnp.float32),
        pltpu.VMEM_SHARED((NP, DEGW), jnp.float32),
        pltpu.SemaphoreType.DMA((KBUF,)),
    ],
)


def _prop_body(hp_hbm, src_hbm, dst_hbm, zeros_hbm, out_hbm,
               agg_sh, gsem, ssem, isem, jsem,
               gi0, gi1, gi2, gi3, di0, di1, di2, di3,
               rv0, rv1, rv2, rv3):
    gidx = (gi0, gi1, gi2, gi3)
    didx = (di0, di1, di2, di3)
    rows = (rv0, rv1, rv2, rv3)
    cid = lax.axis_index("c")
    sid = lax.axis_index("s")
    wid = sid * NC + cid
    r0 = sid * ROWS_PT
    base = wid * EPW
    pltpu.sync_copy(zeros_hbm, agg_sh.at[pl.ds(r0, ROWS_PT)])
    plsc.subcore_barrier()

    def _fetch_g(c, b):
        pltpu.async_copy(
            src_hbm.at[pl.ds(base + c * CHUNK, CHUNK)], gidx[b], isem.at[b])

    def _fetch_d(c, b):
        pltpu.async_copy(
            dst_hbm.at[pl.ds(base + c * CHUNK, CHUNK)], didx[b], jsem.at[b])

    def _gwait_idx(b):
        pltpu.make_async_copy(
            src_hbm.at[pl.ds(base, CHUNK)], gidx[b], isem.at[b]).wait()

    def _dwait_idx(b):
        pltpu.make_async_copy(
            dst_hbm.at[pl.ds(base, CHUNK)], didx[b], jsem.at[b]).wait()

    def _gather(b):
        return pltpu.async_copy(hp_hbm.at[gidx[b]], rows[b], gsem.at[b])

    def _gather_wait(b):
        pltpu.make_async_copy(hp_hbm.at[gidx[b]], rows[b], gsem.at[b]).wait()

    def _scatter(b):
        return pltpu.async_copy(rows[b], agg_sh.at[didx[b]], ssem.at[b],
                                add=True)

    for b in range(PKBUF):
        _fetch_g(b, b)
        _fetch_d(b, b)
    for b in range(PKBUF):
        _gwait_idx(b)
        _gather(b)

    def group(g, carry):
        sd = []
        for b in range(PKBUF):
            nc = lax.rem(g * PKBUF + b + PKBUF, NCHUNK)
            _gather_wait(b)
            _dwait_idx(b)
            sd.append(_scatter(b))
            _fetch_g(nc, b)
        for b in range(PKBUF):
            nc = lax.rem(g * PKBUF + b + PKBUF, NCHUNK)
            sd[b].wait()
            _fetch_d(nc, b)
            _gwait_idx(b)
            _gather(b)
        return carry

    lax.fori_loop(0, PNGROUP, group, 0)
    _gather_wait(0)
    _dwait_idx(0)
    _scatter(0).wait()
    for b in range(1, PKBUF):
        _gather_wait(b)
        _dwait_idx(b)
    plsc.subcore_barrier()
    pltpu.sync_copy(agg_sh.at[pl.ds(r0, ROWS_PT)],
                    out_hbm.at[cid, pl.ds(r0, ROWS_PT)])


_prop_call = pl.kernel(
    _prop_body,
    out_type=jax.ShapeDtypeStruct((NC, NP, H), jnp.float32),
    mesh=_sc_mesh,
    scratch_types=[
        pltpu.VMEM_SHARED((NP, H), jnp.float32),
        pltpu.SemaphoreType.DMA((PKBUF,)),
        pltpu.SemaphoreType.DMA((PKBUF,)),
        pltpu.SemaphoreType.DMA((PKBUF,)),
        pltpu.SemaphoreType.DMA((PKBUF,)),
    ] + [pltpu.VMEM((CHUNK,), jnp.int32) for _ in range(2 * PKBUF)]
      + [pltpu.VMEM((CHUNK, H), jnp.float32) for _ in range(PKBUF)],
)

_RB = 2000
_GRID = N // _RB


def _scale_body(dp_ref, x_ref, w_ref, dis_ref, h_ref):
    dp = dp_ref[...]
    deg = dp[0, :, 0:1] + dp[1, :, 0:1]
    dis = jnp.where(deg > 0, 1.0 / jnp.sqrt(jnp.maximum(deg, 1e-12)), 0.0)
    disb = jnp.broadcast_to(dis, (_RB, H))
    h = jnp.dot(x_ref[...], w_ref[...], preferred_element_type=jnp.float32)
    dis_ref[...] = disb
    h_ref[...] = h * disb


_scale_call = pl.pallas_call(
    _scale_body,
    grid=(_GRID,),
    in_specs=[
        pl.BlockSpec((NC, _RB, DEGW), lambda i: (0, i, 0)),
        pl.BlockSpec((_RB, D), lambda i: (i, 0)),
        pl.BlockSpec((D, H), lambda i: (0, 0)),
    ],
    out_specs=[
        pl.BlockSpec((_RB, H), lambda i: (i, 0)),
        pl.BlockSpec((_RB, H), lambda i: (i, 0)),
    ],
    out_shape=[
        jax.ShapeDtypeStruct((N, H), jnp.float32),
        jax.ShapeDtypeStruct((N, H), jnp.float32),
    ],
)


def _mid_body(p_ref, dis_ref, b_ref, w_ref, out_ref):
    p = p_ref[...]
    dis = dis_ref[...]
    h1 = jnp.maximum(dis * (p[0] + p[1]) + b_ref[...], 0.0)
    out_ref[...] = jnp.dot(
        h1, w_ref[...], preferred_element_type=jnp.float32) * dis


_mid_call = pl.pallas_call(
    _mid_body,
    grid=(_GRID,),
    in_specs=[
        pl.BlockSpec((NC, _RB, H), lambda i: (0, i, 0)),
        pl.BlockSpec((_RB, H), lambda i: (i, 0)),
        pl.BlockSpec((1, H), lambda i: (0, 0)),
        pl.BlockSpec((H, H), lambda i: (0, 0)),
    ],
    out_specs=pl.BlockSpec((_RB, H), lambda i: (i, 0)),
    out_shape=jax.ShapeDtypeStruct((N, H), jnp.float32),
)


def _head_body(p_ref, dis_ref, b_ref, w_ref, bl_ref, out_ref):
    p = p_ref[...]
    h2 = jnp.maximum(dis_ref[...] * (p[0] + p[1]) + b_ref[...], 0.0)
    logits = jnp.dot(
        h2, w_ref[...], preferred_element_type=jnp.float32) + bl_ref[...]
    m = jnp.max(logits, axis=-1, keepdims=True)
    sh = logits - m
    lse = jnp.log(jnp.sum(jnp.exp(sh), axis=-1, keepdims=True))
    out_ref[...] = sh - lse


_head_call = pl.pallas_call(
    _head_body,
    grid=(_GRID,),
    in_specs=[
        pl.BlockSpec((NC, _RB, H), lambda i: (0, i, 0)),
        pl.BlockSpec((_RB, H), lambda i: (i, 0)),
        pl.BlockSpec((1, H), lambda i: (0, 0)),
        pl.BlockSpec((H, C), lambda i: (0, 0)),
        pl.BlockSpec((1, C), lambda i: (0, 0)),
    ],
    out_specs=pl.BlockSpec((_RB, C), lambda i: (i, 0)),
    out_shape=jax.ShapeDtypeStruct((N, C), jnp.float32),
)


def kernel(x, edge_index, batch, W1, b1, W2, b2, Wlin, blin):
    src = edge_index[0]
    dst = edge_index[1]
    dst3 = dst.reshape(NW, NCHUNK, CHUNK)
    zagg = jnp.zeros((ROWS_PT, H), jnp.float32)
    zdeg = jnp.zeros((ROWS_PT, DEGW), jnp.float32)
    ones = jnp.ones((CHUNK, DEGW), jnp.float32)

    deg_parts = _deg_call(dst3, ones, zdeg)
    dis2d, h1p = _scale_call(deg_parts, x, W1)
    p1 = _prop_call(h1p, src, dst, zagg)
    h2p = _mid_call(p1, dis2d, b1.reshape(1, H), W2)
    p2 = _prop_call(h2p, src, dst, zagg)
    return _head_call(p2, dis2d, b2.reshape(1, H), Wlin, blin.reshape(1, C))
